# baseline (speedup 1.0000x reference)
import functools

import jax
import jax.numpy as jnp
from jax import lax
from jax.experimental import pallas as pl
from jax.experimental.pallas import tpu as pltpu
from jax.experimental.pallas import tpu_sc as plsc

H = 512
W = 512
NPH = 32
NPW = 32
PH = H // NPH
PW = W // NPW
NP = NPH * NPW
C = 192

NC = 2
NS = 16
G = 16

ROWF = PW * C
NROWS = NP * PH


def _patch_body(x_hbm, out_hbm, buf0, buf1, sl0, sl1, ss0, ss1):
    wid = lax.axis_index("s") * NC + lax.axis_index("c")
    lane = lax.broadcasted_iota(jnp.int32, (G,), 0)

    def gather(j, buf, sem):
        idx = wid * 512 + j + lane * NPW
        pltpu.async_copy(x_hbm.at[idx], buf, sem)

    def scatter(j, buf, sem):
        odx = wid * 512 + j * PH + lane
        pltpu.async_copy(buf, out_hbm.at[odx], sem)

    def wait_g(buf, sem):
        pltpu.make_async_copy(x_hbm.at[lane], buf, sem).wait()

    def wait_s(buf, sem):
        pltpu.make_async_copy(buf, out_hbm.at[lane], sem).wait()

    gather(0, buf0, sl0)
    gather(1, buf1, sl1)

    def step(i, _):
        j = 2 * i
        wait_g(buf0, sl0)
        scatter(j, buf0, ss0)
        wait_g(buf1, sl1)
        scatter(j + 1, buf1, ss1)
        wait_s(buf0, ss0)
        gather(j + 2, buf0, sl0)
        wait_s(buf1, ss1)
        gather(j + 3, buf1, sl1)
        return 0

    lax.fori_loop(0, (NPW - 2) // 2, step, 0)

    j = NPW - 2
    wait_g(buf0, sl0)
    scatter(j, buf0, ss0)
    wait_g(buf1, sl1)
    scatter(j + 1, buf1, ss1)
    wait_s(buf0, ss0)
    wait_s(buf1, ss1)


_patch_kernel = functools.partial(
    pl.kernel,
    out_type=jax.ShapeDtypeStruct((NROWS, ROWF), jnp.float32),
    mesh=plsc.VectorSubcoreMesh(
        core_axis_name="c", subcore_axis_name="s", num_cores=NC, num_subcores=NS
    ),
    scratch_types=[
        pltpu.VMEM((G, ROWF), jnp.float32),
        pltpu.VMEM((G, ROWF), jnp.float32),
        pltpu.SemaphoreType.DMA,
        pltpu.SemaphoreType.DMA,
        pltpu.SemaphoreType.DMA,
        pltpu.SemaphoreType.DMA,
    ],
)(_patch_body)


def kernel(x, mesh_pos, batch_idx):
    x3 = x.reshape(NROWS, ROWF)
    out = _patch_kernel(x3)
    return out.reshape(1, NP, PH, PW, C)


# speedup vs baseline: 1.7973x; 1.7973x over previous
import functools

import jax
import jax.numpy as jnp
from jax import lax
from jax.experimental import pallas as pl
from jax.experimental.pallas import tpu as pltpu
from jax.experimental.pallas import tpu_sc as plsc

H = 512
W = 512
NPH = 32
NPW = 32
PH = H // NPH
PW = W // NPW
NP = NPH * NPW
C = 192

NC = 2
NS = 16


def _patch_body(x_hbm, out_hbm, buf0, buf1, sl0, sl1, ss0, ss1):
    wid = lax.axis_index("s") * NC + lax.axis_index("c")

    def src(j):
        return x_hbm.at[pl.ds(wid * PH, PH), pl.ds(j * PW, PW), :]

    def dst(j):
        return out_hbm.at[wid * NPW + j]

    pltpu.async_copy(src(0), buf0, sl0)
    pltpu.async_copy(src(1), buf1, sl1)

    def step(i, _):
        j = 2 * i
        pltpu.make_async_copy(src(j), buf0, sl0).wait()
        pltpu.async_copy(buf0, dst(j), ss0)
        pltpu.make_async_copy(src(j + 1), buf1, sl1).wait()
        pltpu.async_copy(buf1, dst(j + 1), ss1)
        pltpu.make_async_copy(buf0, dst(j), ss0).wait()
        pltpu.async_copy(src(j + 2), buf0, sl0)
        pltpu.make_async_copy(buf1, dst(j + 1), ss1).wait()
        pltpu.async_copy(src(j + 3), buf1, sl1)
        return 0

    lax.fori_loop(0, (NPW - 2) // 2, step, 0)

    j = NPW - 2
    pltpu.make_async_copy(src(j), buf0, sl0).wait()
    pltpu.async_copy(buf0, dst(j), ss0)
    pltpu.make_async_copy(src(j + 1), buf1, sl1).wait()
    pltpu.async_copy(buf1, dst(j + 1), ss1)
    pltpu.make_async_copy(buf0, dst(j), ss0).wait()
    pltpu.make_async_copy(buf1, dst(j + 1), ss1).wait()


_patch_kernel = functools.partial(
    pl.kernel,
    out_type=jax.ShapeDtypeStruct((NP, PH, PW, C), jnp.float32),
    mesh=plsc.VectorSubcoreMesh(
        core_axis_name="c", subcore_axis_name="s", num_cores=NC, num_subcores=NS
    ),
    scratch_types=[
        pltpu.VMEM((PH, PW, C), jnp.float32),
        pltpu.VMEM((PH, PW, C), jnp.float32),
        pltpu.SemaphoreType.DMA,
        pltpu.SemaphoreType.DMA,
        pltpu.SemaphoreType.DMA,
        pltpu.SemaphoreType.DMA,
    ],
    compiler_params=pltpu.CompilerParams(use_tc_tiling_on_sc=True),
)(_patch_body)


def kernel(x, mesh_pos, batch_idx):
    x3 = x.reshape(H, W, C)
    out = _patch_kernel(x3)
    return out.reshape(1, NP, PH, PW, C)


# speedup vs baseline: 1.8067x; 1.0052x over previous
import functools

import jax
import jax.numpy as jnp
from jax import lax
from jax.experimental import pallas as pl
from jax.experimental.pallas import tpu as pltpu
from jax.experimental.pallas import tpu_sc as plsc

H = 512
W = 512
NPH = 32
NPW = 32
PH = H // NPH
PW = W // NPW
NP = NPH * NPW
C = 192

NC = 2
NS = 16
HP = PH // 2
NCH = NPW * 2
NBUF = 4


def _patch_body(x_hbm, out_hbm, bufs, sls, sss):
    wid = lax.axis_index("s") * NC + lax.axis_index("c")

    def src(ch):
        j = lax.div(ch, 2)
        jh = lax.rem(ch, 2)
        return x_hbm.at[
            pl.ds(wid * PH + jh * HP, HP), pl.ds(j * PW, PW), :
        ]

    def dst(ch):
        j = lax.div(ch, 2)
        jh = lax.rem(ch, 2)
        return out_hbm.at[wid * NPW + j, pl.ds(jh * HP, HP)]

    def load(ch, b):
        pltpu.async_copy(src(ch), bufs[b], sls[b])

    def store(ch, b):
        pltpu.async_copy(bufs[b], dst(ch), sss[b])

    def wait_l(b):
        pltpu.make_async_copy(src(0), bufs[b], sls[b]).wait()

    def wait_s(b):
        pltpu.make_async_copy(bufs[b], dst(0), sss[b]).wait()

    for b in range(NBUF):
        load(b, b)

    def step(i, _):
        ch = NBUF * i
        for b in range(NBUF):
            wait_l(b)
            store(ch + b, b)
        for b in range(NBUF):
            wait_s(b)
            load(ch + NBUF + b, b)
        return 0

    lax.fori_loop(0, NCH // NBUF - 1, step, 0)

    ch = NCH - NBUF
    for b in range(NBUF):
        wait_l(b)
        store(ch + b, b)
    for b in range(NBUF):
        wait_s(b)


_patch_kernel = functools.partial(
    pl.kernel,
    out_type=jax.ShapeDtypeStruct((NP, PH, PW, C), jnp.float32),
    mesh=plsc.VectorSubcoreMesh(
        core_axis_name="c", subcore_axis_name="s", num_cores=NC, num_subcores=NS
    ),
    scratch_types=[
        [pltpu.VMEM((HP, PW, C), jnp.float32) for _ in range(NBUF)],
        [pltpu.SemaphoreType.DMA for _ in range(NBUF)],
        [pltpu.SemaphoreType.DMA for _ in range(NBUF)],
    ],
    compiler_params=pltpu.CompilerParams(use_tc_tiling_on_sc=True),
)(_patch_body)


def kernel(x, mesh_pos, batch_idx):
    x3 = x.reshape(H, W, C)
    out = _patch_kernel(x3)
    return out.reshape(1, NP, PH, PW, C)


# speedup vs baseline: 1.8471x; 1.0224x over previous
import functools

import jax
import jax.numpy as jnp
from jax import lax
from jax.experimental import pallas as pl
from jax.experimental.pallas import tpu as pltpu
from jax.experimental.pallas import tpu_sc as plsc

H = 512
W = 512
NPH = 32
NPW = 32
PH = H // NPH
PW = W // NPW
NP = NPH * NPW
C = 192

NC = 2
NS = 16
HP = PH // 2
NBUF = 4

NPH_SC = 16
TPS = NC * NS // NPH_SC
NCH = (NPW // TPS) * 2


def _sc_body(x_hbm, out_hbm, bufs, sls, sss):
    wid = lax.axis_index("s") * NC + lax.axis_index("c")
    nph = lax.div(wid, TPS)
    half = lax.rem(wid, TPS)

    def src(ch):
        j = half * (NPW // TPS) + lax.div(ch, 2)
        jh = lax.rem(ch, 2)
        return x_hbm.at[
            pl.ds(nph * PH + jh * HP, HP), pl.ds(j * PW, PW), :
        ]

    def dst(ch):
        j = half * (NPW // TPS) + lax.div(ch, 2)
        jh = lax.rem(ch, 2)
        return out_hbm.at[nph * NPW + j, pl.ds(jh * HP, HP)]

    def load(ch, b):
        pltpu.async_copy(src(ch), bufs[b], sls[b])

    def store(ch, b):
        pltpu.async_copy(bufs[b], dst(ch), sss[b])

    def wait_l(b):
        pltpu.make_async_copy(src(0), bufs[b], sls[b]).wait()

    def wait_s(b):
        pltpu.make_async_copy(bufs[b], dst(0), sss[b]).wait()

    for b in range(NBUF):
        load(b, b)

    def step(i, _):
        ch = NBUF * i
        for b in range(NBUF):
            wait_l(b)
            store(ch + b, b)
        for b in range(NBUF):
            wait_s(b)
            load(ch + NBUF + b, b)
        return 0

    lax.fori_loop(0, NCH // NBUF - 1, step, 0)

    ch = NCH - NBUF
    for b in range(NBUF):
        wait_l(b)
        store(ch + b, b)
    for b in range(NBUF):
        wait_s(b)


_sc_kernel = functools.partial(
    pl.kernel,
    out_type=jax.ShapeDtypeStruct((NP, PH, PW, C), jnp.float32),
    mesh=plsc.VectorSubcoreMesh(
        core_axis_name="c", subcore_axis_name="s", num_cores=NC, num_subcores=NS
    ),
    scratch_types=[
        [pltpu.VMEM((HP, PW, C), jnp.float32) for _ in range(NBUF)],
        [pltpu.SemaphoreType.DMA for _ in range(NBUF)],
        [pltpu.SemaphoreType.DMA for _ in range(NBUF)],
    ],
    compiler_params=pltpu.CompilerParams(use_tc_tiling_on_sc=True),
)(_sc_body)


def _tc_body(x_ref, y_ref, o_ref):
    blk = x_ref[...].reshape(PH, NPW, PW, C)
    o_ref[...] = jnp.swapaxes(blk, 0, 1)


def _tc_kernel(x3, y):
    return pl.pallas_call(
        _tc_body,
        grid=(NPH - NPH_SC,),
        in_specs=[
            pl.BlockSpec((PH, W, C), lambda i: (NPH_SC + i, 0, 0)),
            pl.BlockSpec(memory_space=pl.ANY),
        ],
        out_specs=pl.BlockSpec(
            (NPW, PH, PW, C), lambda i: (NPH_SC + i, 0, 0, 0)
        ),
        out_shape=jax.ShapeDtypeStruct((NP, PH, PW, C), jnp.float32),
        input_output_aliases={1: 0},
    )(x3, y)


def kernel(x, mesh_pos, batch_idx):
    x3 = x.reshape(H, W, C)
    out = _sc_kernel(x3)
    out = _tc_kernel(x3, out)
    return out.reshape(1, NP, PH, PW, C)
